# baseline (speedup 1.0000x reference)
import jax
import jax.numpy as jnp
from jax.experimental import pallas as pl
from jax.experimental.pallas import tpu as pltpu

_INPUT_DIM = 1024
_INTER_DIM = 2048
_GATE_NUM = 8
_TOP_P = 0.3

_IT = 512
_TM = 512
_C = _INTER_DIM // _IT


def _fused_moe_kernel(x_ref, gate_w_ref, ew_ref, ow_ref, o_ref,
                      acc_ref, ab_ref, xs_ref):
    j = pl.program_id(0)

    @pl.when(j < _C)
    def _prep():
        w0 = ew_ref[0].astype(jnp.bfloat16)
        w1 = ew_ref[1].astype(jnp.bfloat16)
        ow = ow_ref[...].astype(jnp.bfloat16)
        pa0 = jax.lax.dot_general(
            ow, w0, dimension_numbers=(((1,), (0,)), ((), ())),
            preferred_element_type=jnp.float32)
        pa1 = jax.lax.dot_general(
            ow, w1, dimension_numbers=(((1,), (0,)), ((), ())),
            preferred_element_type=jnp.float32)

        @pl.when(j == 0)
        def _init():
            acc_ref[:, :_INPUT_DIM] = pa0
            acc_ref[:, _INPUT_DIM:] = pa1

        @pl.when(j > 0)
        def _acc():
            acc_ref[:, :_INPUT_DIM] += pa0
            acc_ref[:, _INPUT_DIM:] += pa1

        @pl.when(j == _C - 1)
        def _cast():
            ab_ref[...] = acc_ref[...].astype(jnp.bfloat16)

        x = x_ref[...]
        g = jax.lax.dot_general(
            x, gate_w_ref[...],
            dimension_numbers=(((1,), (1,)), ((), ())),
            preferred_element_type=jnp.float32,
        )
        m = jnp.max(g, axis=1, keepdims=True)
        e = jnp.exp(g - m)
        probs = e / jnp.sum(e, axis=1, keepdims=True)
        c1 = jnp.sum((probs > _TOP_P).astype(jnp.float32), axis=1,
                     keepdims=True)
        s0 = probs[:, 0:1] * (_GATE_NUM - c1)
        s1 = probs[:, 1:2] * c1
        xs_ref[pl.ds(j * _TM, _TM), :_INPUT_DIM] = (x * s0).astype(jnp.bfloat16)
        xs_ref[pl.ds(j * _TM, _TM), _INPUT_DIM:] = (x * s1).astype(jnp.bfloat16)

    @pl.when(j >= _C)
    def _tokens():
        t = j - _C
        o_ref[...] = jax.lax.dot_general(
            xs_ref[pl.ds(t * _TM, _TM), :], ab_ref[...],
            dimension_numbers=(((1,), (1,)), ((), ())),
            preferred_element_type=jnp.float32,
        )


def kernel(x, gate_W, expert_W, out_W):
    bsz, seql, embs = x.shape
    n = bsz * seql
    x_flat = x.reshape(n, embs)
    n_token_tiles = n // _TM

    grid = (_C + n_token_tiles,)
    out = pl.pallas_call(
        _fused_moe_kernel,
        grid=grid,
        in_specs=[
            pl.BlockSpec((_TM, embs),
                         lambda j: (jnp.minimum(j, _C - 1), 0)),
            pl.BlockSpec((_GATE_NUM, embs), lambda j: (0, 0)),
            pl.BlockSpec((2, _IT, embs),
                         lambda j: (0, jnp.minimum(j, _C - 1), 0)),
            pl.BlockSpec((embs, _IT),
                         lambda j: (0, jnp.minimum(j, _C - 1))),
        ],
        out_specs=pl.BlockSpec((_TM, embs),
                               lambda j: (jnp.maximum(j - _C, 0), 0)),
        out_shape=jax.ShapeDtypeStruct((n, embs), jnp.float32),
        scratch_shapes=[
            pltpu.VMEM((_INPUT_DIM, 2 * _INPUT_DIM), jnp.float32),
            pltpu.VMEM((_INPUT_DIM, 2 * _INPUT_DIM), jnp.bfloat16),
            pltpu.VMEM((2048, 2 * _INPUT_DIM), jnp.bfloat16),
        ],
        compiler_params=pltpu.CompilerParams(
            dimension_semantics=("arbitrary",),
        ),
    )(x_flat, gate_W, expert_W, out_W)
    return out.reshape(bsz, seql, embs)


# speedup vs baseline: 1.1471x; 1.1471x over previous
import jax
import jax.numpy as jnp
from jax.experimental import pallas as pl
from jax.experimental.pallas import tpu as pltpu

_INPUT_DIM = 1024
_INTER_DIM = 2048
_GATE_NUM = 8
_TOP_P = 0.3

_DS = 512
_TM = 1024
_C = _INPUT_DIM // _DS


def _fused_moe_kernel(x_ref, gate_w_ref, ew_ref, ow_ref, o_ref,
                      owb_ref, ab_ref):
    j = pl.program_id(0)

    @pl.when(j == 0)
    def _cast_ow():
        owb_ref[...] = ow_ref[...].astype(jnp.bfloat16)

    @pl.when(j < _C)
    def _build_a_slab():
        w0 = ew_ref[0].astype(jnp.bfloat16)
        w1 = ew_ref[1].astype(jnp.bfloat16)
        ow = owb_ref[...]
        ab_ref[:, pl.ds(j * _DS, _DS)] = jax.lax.dot_general(
            ow, w0, dimension_numbers=(((1,), (0,)), ((), ())),
            preferred_element_type=jnp.float32).astype(jnp.bfloat16)
        ab_ref[:, pl.ds(_INPUT_DIM + j * _DS, _DS)] = jax.lax.dot_general(
            ow, w1, dimension_numbers=(((1,), (0,)), ((), ())),
            preferred_element_type=jnp.float32).astype(jnp.bfloat16)

    @pl.when(j >= _C)
    def _tokens():
        x = x_ref[...]
        g = jax.lax.dot_general(
            x, gate_w_ref[...],
            dimension_numbers=(((1,), (1,)), ((), ())),
            preferred_element_type=jnp.float32,
        )
        m = jnp.max(g, axis=1, keepdims=True)
        e = jnp.exp(g - m)
        probs = e / jnp.sum(e, axis=1, keepdims=True)
        c1 = jnp.sum((probs > _TOP_P).astype(jnp.float32), axis=1,
                     keepdims=True)
        s0 = probs[:, 0:1] * (_GATE_NUM - c1)
        s1 = probs[:, 1:2] * c1
        x01 = jnp.concatenate(
            [(x * s0).astype(jnp.bfloat16), (x * s1).astype(jnp.bfloat16)],
            axis=1)
        o_ref[...] = jax.lax.dot_general(
            x01, ab_ref[...],
            dimension_numbers=(((1,), (1,)), ((), ())),
            preferred_element_type=jnp.float32,
        )


def kernel(x, gate_W, expert_W, out_W):
    bsz, seql, embs = x.shape
    n = bsz * seql
    x_flat = x.reshape(n, embs)
    n_token_tiles = n // _TM

    grid = (_C + n_token_tiles,)
    out = pl.pallas_call(
        _fused_moe_kernel,
        grid=grid,
        in_specs=[
            pl.BlockSpec((_TM, embs),
                         lambda j: (jnp.maximum(j - _C, 0), 0)),
            pl.BlockSpec((_GATE_NUM, embs), lambda j: (0, 0)),
            pl.BlockSpec((2, _INTER_DIM, _DS),
                         lambda j: (0, 0, jnp.minimum(j, _C - 1))),
            pl.BlockSpec((embs, _INTER_DIM), lambda j: (0, 0)),
        ],
        out_specs=pl.BlockSpec((_TM, embs),
                               lambda j: (jnp.maximum(j - _C, 0), 0)),
        out_shape=jax.ShapeDtypeStruct((n, embs), jnp.float32),
        scratch_shapes=[
            pltpu.VMEM((_INPUT_DIM, _INTER_DIM), jnp.bfloat16),
            pltpu.VMEM((_INPUT_DIM, 2 * _INPUT_DIM), jnp.bfloat16),
        ],
        compiler_params=pltpu.CompilerParams(
            dimension_semantics=("arbitrary",),
        ),
    )(x_flat, gate_W, expert_W, out_W)
    return out.reshape(bsz, seql, embs)
